# baseline (speedup 1.0000x reference)
import functools

import jax
import jax.numpy as jnp
from jax import lax
from jax.experimental import pallas as pl
from jax.experimental.pallas import tpu as pltpu
from jax.experimental.pallas import tpu_sc as plsc

F32 = jnp.float32
I32 = jnp.int32

N = 100000
E = 1600000
B = 2048
F = 48
L = 16
NCH = F // L
EPS = 1e-5

NC, NS = 2, 16
NW = NC * NS

K_SUB = 8
UNIT = 512
EROWS_P = -(-E // (128 * K_SUB * NW)) * (K_SUB * NW)
PAD_E = EROWS_P * 128 - E
E_UNITS = EROWS_P * 128 // UNIT
UNITS_PER_W = E_UNITS // NW
NP = N + 8

SEG_W = 6248
SEG_TAIL = N - SEG_W * NS
CNT_W = 6240
CNT_TAIL = N - CNT_W * NS
CQ = 2080

PROWS = N // 128
PTAIL = N - PROWS * 128
PROWS_P = 784
PGROUPS = PROWS_P // 8
PGPW = 3

R_TC = 2000
GRID_TC = N // R_TC

_MESH = plsc.VectorSubcoreMesh(core_axis_name="c", subcore_axis_name="s")
_SC_PARAMS = pltpu.CompilerParams(use_tc_tiling_on_sc=False)
_SC_PARAMS_POOL = pltpu.CompilerParams(use_tc_tiling_on_sc=False,
                                       needs_layout_passes=False)


def _relu(v):
    return jnp.maximum(v, 0.0)


def _dot(a, b):
    return jnp.dot(a, b, preferred_element_type=F32)


def _make_seg_sum(with_cnt):
    outs = [jax.ShapeDtypeStruct((NC, N, F), F32)]
    if with_cnt:
        outs.append(jax.ShapeDtypeStruct((NC * N,), F32))
    scratch = [
        pltpu.VMEM((4, UNIT), I32),
        pltpu.VMEM((4, UNIT), I32),
        pltpu.VMEM((2, UNIT, L), F32),
        pltpu.VMEM((UNIT,), F32),
        pltpu.VMEM((CQ,), F32),
        pltpu.VMEM_SHARED((NP, L), F32),
        pltpu.SemaphoreType.DMA,
        pltpu.SemaphoreType.DMA,
        pltpu.SemaphoreType.DMA,
        pltpu.SemaphoreType.DMA,
        pltpu.SemaphoreType.DMA,
        pltpu.SemaphoreType.DMA,
    ]
    if with_cnt:
        scratch.append(pltpu.VMEM_SHARED((NP,), F32))

    @functools.partial(pl.kernel, out_type=tuple(outs), mesh=_MESH,
                       scratch_types=scratch, compiler_params=_SC_PARAMS)
    def seg(t0, t1, t2, src_h, dst_h, z2d, z1d, *rest):
        o = rest[0]
        if with_cnt:
            ocnt = rest[1]
            (sv3, dv3, rows, ones, zc, acc,
             ssem0, ssem1, isem0, isem1, gsem0, gsem1, cnt) = rest[2:15]
        else:
            (sv3, dv3, rows, ones, zc, acc,
             ssem0, ssem1, isem0, isem1, gsem0, gsem1) = rest[1:13]
        ssem = (ssem0, ssem1)
        isem = (isem0, isem1)
        gsem = (gsem0, gsem1)
        tabs = (t0, t1, t2)
        cid = lax.axis_index("c")
        sid = lax.axis_index("s")
        wid = cid * NS + sid

        if with_cnt:
            for i in range(UNIT // L):
                ones[pl.ds(i * L, L)] = jnp.ones((L,), F32)

        seg_chunks = []
        off = 0
        while off < SEG_W:
            sz = min(UNIT, SEG_W - off)
            seg_chunks.append((off, sz))
            off += sz

        def edge_pipeline(tab, base, do_cnt):
            NU = UNITS_PER_W

            def drain_idx(p):
                pltpu.make_async_copy(src_h.at[0], sv3.at[0],
                                      isem[p]).wait()
                pltpu.make_async_copy(src_h.at[0], dv3.at[0],
                                      isem[p]).wait()

            def drain_scatter(p):
                pltpu.make_async_copy(z2d, rows.at[p], ssem[p]).wait()

            def unit_step(k, u, guard_early):
                p = u % 2
                if guard_early:
                    @pl.when(k >= 2)
                    def _drains():
                        drain_scatter(p)
                        drain_idx(p)
                else:
                    drain_scatter(p)
                    drain_idx(p)
                desc = pltpu.async_copy(tab.at[sv3.at[u]], rows.at[p],
                                        gsem[p])
                @pl.when(k + 2 < NU)
                def _pf():
                    r1 = base + k + 2
                    pltpu.async_copy(src_h.at[r1],
                                     sv3.at[(u + 2) % 4], isem[p])
                    pltpu.async_copy(dst_h.at[r1],
                                     dv3.at[(u + 2) % 4], isem[p])
                desc.wait()
                pltpu.async_copy(rows.at[p], acc.at[dv3.at[u]], ssem[p],
                                 add=True)
                if do_cnt:
                    pltpu.sync_copy(ones, cnt.at[dv3.at[u]], add=True)

            pltpu.sync_copy(src_h.at[base], sv3.at[0])
            pltpu.sync_copy(dst_h.at[base], dv3.at[0])
            pltpu.sync_copy(src_h.at[base + 1], sv3.at[1])
            pltpu.sync_copy(dst_h.at[base + 1], dv3.at[1])

            def body(i, carry):
                for u in range(4):
                    unit_step(i * 4 + u, u, guard_early=(u < 2))
                return carry

            lax.fori_loop(0, NU // 4, body, 0)
            for u in range(2):
                unit_step((NU // 4) * 4 + u, u, guard_early=False)
            drain_scatter(0)
            drain_scatter(1)

        def zero_acc():
            pltpu.sync_copy(z2d, rows.at[0])
            for off, sz in seg_chunks:
                pltpu.sync_copy(rows.at[0, pl.ds(0, sz)],
                                acc.at[pl.ds(sid * SEG_W + off, sz)])

            @pl.when(sid == 0)
            def _zt():
                pltpu.sync_copy(rows.at[0, pl.ds(0, SEG_TAIL)],
                                acc.at[pl.ds(NS * SEG_W, SEG_TAIL)])

        def edge_pass(fc, do_cnt):
            edge_pipeline(tabs[fc], wid * UNITS_PER_W, do_cnt)

        zero_acc()
        if with_cnt:
            pltpu.sync_copy(z1d, zc)
            for q in range(CNT_W // CQ):
                pltpu.sync_copy(zc,
                                cnt.at[pl.ds(sid * CNT_W + q * CQ, CQ)])

            @pl.when(sid == 0)
            def _zct():
                pltpu.sync_copy(zc.at[pl.ds(0, CNT_TAIL)],
                                cnt.at[pl.ds(NS * CNT_W, CNT_TAIL)])
        plsc.subcore_barrier()

        for fc in range(NCH):
            edge_pass(fc, with_cnt and fc == 0)
            plsc.subcore_barrier()
            for off, sz in seg_chunks:
                a0 = sid * SEG_W + off
                pltpu.sync_copy(acc.at[pl.ds(a0, sz)],
                                rows.at[0, pl.ds(0, sz)])
                pltpu.sync_copy(rows.at[0, pl.ds(0, sz)],
                                o.at[cid, pl.ds(a0, sz), pl.ds(fc * L, L)])

            @pl.when(sid == 0)
            def _wtail():
                pltpu.sync_copy(acc.at[pl.ds(NS * SEG_W, SEG_TAIL)],
                                rows.at[0, pl.ds(0, SEG_TAIL)])
                pltpu.sync_copy(
                    rows.at[0, pl.ds(0, SEG_TAIL)],
                    o.at[cid, pl.ds(NS * SEG_W, SEG_TAIL), pl.ds(fc * L, L)])

            if with_cnt and fc == 0:
                for q in range(CNT_W // CQ):
                    c0 = sid * CNT_W + q * CQ
                    pltpu.sync_copy(cnt.at[pl.ds(c0, CQ)], zc)
                    pltpu.sync_copy(zc, ocnt.at[pl.ds(cid * N + c0, CQ)])

                @pl.when(sid == 0)
                def _wt():
                    pltpu.sync_copy(cnt.at[pl.ds(NS * CNT_W, CNT_TAIL)],
                                    zc.at[pl.ds(0, CNT_TAIL)])
                    pltpu.sync_copy(
                        zc.at[pl.ds(0, CNT_TAIL)],
                        ocnt.at[pl.ds(cid * N + NS * CNT_W, CNT_TAIL)])
            plsc.subcore_barrier()
            if fc + 1 < NCH:
                zero_acc()
                plsc.subcore_barrier()

    return seg


_seg_sum_cnt = _make_seg_sum(True)
_seg_sum = _make_seg_sum(False)


@functools.partial(
    pl.kernel,
    out_type=(jax.ShapeDtypeStruct((NC, B, F), F32),
              jax.ShapeDtypeStruct((NC * B,), F32),
              jax.ShapeDtypeStruct((NW, B, F), F32)),
    mesh=_MESH,
    compiler_params=_SC_PARAMS_POOL,
    scratch_types=[
        pltpu.VMEM((128, F), F32),
        pltpu.VMEM((8, 128), I32),
        pltpu.VMEM((PTAIL, F), F32),
        pltpu.VMEM((1, PTAIL), I32),
        pltpu.VMEM((128,), F32),
        pltpu.VMEM((128,), F32),
        pltpu.VMEM((B, F), F32),
        pltpu.VMEM_SHARED((B, F), F32),
        pltpu.VMEM_SHARED((B,), F32),
    ])
def _pool_sc(h_h, b2d_h, btail_h, zslab, zcnt, ogsum, ogcnt, ogmax,
             rows, ids8, rowst, idt, ones, pcbuf, slab, gsum, gcnt):
    cid = lax.axis_index("c")
    sid = lax.axis_index("s")
    wid = cid * NS + sid

    for i in range(128 // L):
        ones[pl.ds(i * L, L)] = jnp.ones((L,), F32)

    pltpu.sync_copy(zslab, slab)
    pltpu.sync_copy(zcnt, pcbuf)
    pltpu.sync_copy(slab.at[pl.ds(0, 128)], gsum.at[pl.ds(sid * 128, 128)])
    pltpu.sync_copy(pcbuf, gcnt.at[pl.ds(sid * 128, 128)])
    plsc.subcore_barrier()

    zero16 = jnp.zeros((L,), I32)

    def max_rows(rows_ref, ids_ref, idrow, nrows):
        rr8 = jnp.full((L,), idrow, I32)

        def mrow(r, carry):
            rr = jnp.full((L,), r, I32)
            bid = plsc.load_gather(ids_ref, [rr8, rr])
            for fc in range(NCH):
                col = fc * L + lax.iota(I32, L)
                v = plsc.load_gather(rows_ref, [rr, col])
                old = plsc.load_gather(slab, [bid, col])
                plsc.store_scatter(slab, [bid, col], jnp.maximum(old, v))
            return carry

        lax.fori_loop(0, nrows, mrow, 0)

    def do_group(g, nrows):
        pltpu.sync_copy(b2d_h.at[pl.ds(g * 8, 8)], ids8)
        for r8 in range(nrows):
            row = g * 8 + r8
            pltpu.sync_copy(h_h.at[pl.ds(row * 128, 128)], rows)
            pltpu.sync_copy(rows, gsum.at[ids8.at[r8]], add=True)
            pltpu.sync_copy(ones, gcnt.at[ids8.at[r8]], add=True)
            max_rows(rows, ids8, r8, 128)

    base = wid * PGPW

    def body(k, carry):
        do_group(base + k, 8)
        return carry

    lax.fori_loop(0, PGPW, body, 0)

    @pl.when(wid == 0)
    def _extra0():
        do_group(NW * PGPW, 8)

    @pl.when(wid == 1)
    def _extra1():
        do_group(NW * PGPW + 1, PROWS - (NW * PGPW + 1) * 8)

    @pl.when(wid == 0)
    def _tail():
        pltpu.sync_copy(btail_h, idt)
        pltpu.sync_copy(h_h.at[pl.ds(PROWS * 128, PTAIL)], rowst)
        pltpu.sync_copy(rowst, gsum.at[idt.at[0]], add=True)
        pltpu.sync_copy(ones.at[pl.ds(0, PTAIL)], gcnt.at[idt.at[0]],
                        add=True)
        max_rows(rowst, idt, 0, PTAIL)

    plsc.subcore_barrier()
    pltpu.sync_copy(slab, ogmax.at[wid])
    pltpu.sync_copy(gsum.at[pl.ds(sid * 128, 128)], rows)
    pltpu.sync_copy(rows, ogsum.at[cid, pl.ds(sid * 128, 128)])
    pltpu.sync_copy(gcnt.at[pl.ds(sid * 128, 128)], pcbuf)
    pltpu.sync_copy(pcbuf, ogcnt.at[pl.ds(cid * B + sid * 128, 128)])


def _proj_body(x_ref, wl_ref, wr_ref, y0_ref, y1_ref, y2_ref, w_ref):
    xb = x_ref[...]
    a = _dot(xb, wl_ref[...])
    y0_ref[...] = a[:, 0 * L:1 * L]
    y1_ref[...] = a[:, 1 * L:2 * L]
    y2_ref[...] = a[:, 2 * L:3 * L]
    w_ref[...] = _dot(xb, wr_ref[...])


def _proj(x, wl, wr):
    return pl.pallas_call(
        _proj_body,
        grid=(GRID_TC,),
        in_specs=[
            pl.BlockSpec((R_TC, F), lambda i: (i, 0)),
            pl.BlockSpec((F, F), lambda i: (0, 0)),
            pl.BlockSpec((F, F), lambda i: (0, 0)),
        ],
        out_specs=[
            pl.BlockSpec((R_TC, L), lambda i: (i, 0)),
            pl.BlockSpec((R_TC, L), lambda i: (i, 0)),
            pl.BlockSpec((R_TC, L), lambda i: (i, 0)),
            pl.BlockSpec((R_TC, F), lambda i: (i, 0)),
        ],
        out_shape=[
            jax.ShapeDtypeStruct((N, L), F32),
            jax.ShapeDtypeStruct((N, L), F32),
            jax.ShapeDtypeStruct((N, L), F32),
            jax.ShapeDtypeStruct((N, F), F32),
        ],
    )(x, wl, wr)


def _make_pre_stats(first):
    def body(*refs):
        if first:
            (s_ref, cnt_ref, w_ref, b_ref,
             pre_ref, stats_ref, cvec_ref) = refs
            c = jnp.maximum(cnt_ref[0] + cnt_ref[1], 1.0)
            cvec_ref[...] = c
        else:
            (s_ref, cvec_ref, w_ref, b_ref,
             pre_ref, stats_ref) = refs
            c = cvec_ref[...]
        s = s_ref[0] + s_ref[1]
        pre = s / c + b_ref[...] + w_ref[...]
        pre_ref[...] = pre

        @pl.when(pl.program_id(0) == 0)
        def _init():
            stats_ref[...] = jnp.zeros((2, F), F32)

        st = jnp.concatenate(
            [jnp.sum(pre, axis=0, keepdims=True),
             jnp.sum(pre * pre, axis=0, keepdims=True)], axis=0)
        stats_ref[...] += st

    sp_spec = pl.BlockSpec((NC, R_TC, F), lambda i: (0, i, 0))
    cspec = pl.BlockSpec((NC, R_TC, 1), lambda i: (0, i, 0)) if first else \
        pl.BlockSpec((R_TC, 1), lambda i: (i, 0))
    out_specs = [
        pl.BlockSpec((R_TC, F), lambda i: (i, 0)),
        pl.BlockSpec((2, F), lambda i: (0, 0)),
    ]
    out_shape = [
        jax.ShapeDtypeStruct((N, F), F32),
        jax.ShapeDtypeStruct((2, F), F32),
    ]
    if first:
        out_specs.append(pl.BlockSpec((R_TC, 1), lambda i: (i, 0)))
        out_shape.append(jax.ShapeDtypeStruct((N, 1), F32))

    def run(s, cdata, w, bias):
        return pl.pallas_call(
            body,
            grid=(GRID_TC,),
            in_specs=[sp_spec, cspec,
                      pl.BlockSpec((R_TC, F), lambda i: (i, 0)),
                      pl.BlockSpec((1, F), lambda i: (0, 0))],
            out_specs=out_specs,
            out_shape=out_shape,
        )(s, cdata, w, bias)

    return run


_pre_stats_first = _make_pre_stats(True)
_pre_stats_next = _make_pre_stats(False)


def _bn_common(pre_ref, stats_ref, g_ref, be_ref):
    st = stats_ref[...]
    mean = st[0:1, :] * (1.0 / N)
    var = st[1:2, :] * (1.0 / N) - mean * mean
    rstd = lax.rsqrt(var + EPS)
    return _relu((pre_ref[...] - mean) * rstd * g_ref[...] + be_ref[...])


def _bn_proj_body(pre_ref, stats_ref, g_ref, be_ref, wl_ref, wr_ref,
                  y0_ref, y1_ref, y2_ref, w_ref):
    h = _bn_common(pre_ref, stats_ref, g_ref, be_ref)
    a = _dot(h, wl_ref[...])
    y0_ref[...] = a[:, 0 * L:1 * L]
    y1_ref[...] = a[:, 1 * L:2 * L]
    y2_ref[...] = a[:, 2 * L:3 * L]
    w_ref[...] = _dot(h, wr_ref[...])


def _bn_proj(pre, stats, g, be, wl, wr):
    return pl.pallas_call(
        _bn_proj_body,
        grid=(GRID_TC,),
        in_specs=[
            pl.BlockSpec((R_TC, F), lambda i: (i, 0)),
            pl.BlockSpec((2, F), lambda i: (0, 0)),
            pl.BlockSpec((1, F), lambda i: (0, 0)),
            pl.BlockSpec((1, F), lambda i: (0, 0)),
            pl.BlockSpec((F, F), lambda i: (0, 0)),
            pl.BlockSpec((F, F), lambda i: (0, 0)),
        ],
        out_specs=[
            pl.BlockSpec((R_TC, L), lambda i: (i, 0)),
            pl.BlockSpec((R_TC, L), lambda i: (i, 0)),
            pl.BlockSpec((R_TC, L), lambda i: (i, 0)),
            pl.BlockSpec((R_TC, F), lambda i: (i, 0)),
        ],
        out_shape=[
            jax.ShapeDtypeStruct((N, L), F32),
            jax.ShapeDtypeStruct((N, L), F32),
            jax.ShapeDtypeStruct((N, L), F32),
            jax.ShapeDtypeStruct((N, F), F32),
        ],
    )(pre, stats, g, be, wl, wr)


def _bn_relu_body(pre_ref, stats_ref, g_ref, be_ref, h_ref):
    h_ref[...] = _bn_common(pre_ref, stats_ref, g_ref, be_ref)


def _bn_relu(pre, stats, g, be):
    return pl.pallas_call(
        _bn_relu_body,
        grid=(GRID_TC,),
        in_specs=[
            pl.BlockSpec((R_TC, F), lambda i: (i, 0)),
            pl.BlockSpec((2, F), lambda i: (0, 0)),
            pl.BlockSpec((1, F), lambda i: (0, 0)),
            pl.BlockSpec((1, F), lambda i: (0, 0)),
        ],
        out_specs=pl.BlockSpec((R_TC, F), lambda i: (i, 0)),
        out_shape=jax.ShapeDtypeStruct((N, F), F32),
    )(pre, stats, g, be)


def _head_body(gsum_ref, gcnt_ref, gmax_ref, adme_ref,
               wh1_ref, bh1_ref, gh_ref, beh_ref, wh2_ref, bh2_ref,
               wh3_ref, bh3_ref, out_ref):
    gsum = gsum_ref[0] + gsum_ref[1]
    gcnt = jnp.maximum(gcnt_ref[0] + gcnt_ref[1], 1.0)
    gmean = gsum / gcnt
    m = gmax_ref[0]
    for i in range(1, NW):
        m = jnp.maximum(m, gmax_ref[i])
    comb = jnp.concatenate([gmean, m, adme_ref[...]], axis=-1)
    z = _dot(comb, wh1_ref[...]) + bh1_ref[...]
    mean = jnp.mean(z, axis=0, keepdims=True)
    var = jnp.mean(z * z, axis=0, keepdims=True) - mean * mean
    z = _relu((z - mean) * lax.rsqrt(var + EPS) * gh_ref[...] + beh_ref[...])
    z = _relu(_dot(z, wh2_ref[...]) + bh2_ref[...])
    out_ref[...] = _dot(z, wh3_ref[...]) + bh3_ref[...]


def _head(gsum, gcnt, gmax, adme, wh1, bh1, gh, beh, wh2, bh2, wh3, bh3):
    full = lambda shape: pl.BlockSpec(shape, lambda: tuple(0 for _ in shape))
    ins = [gsum, gcnt, gmax, adme, wh1, bh1, gh, beh, wh2, bh2, wh3, bh3]
    return pl.pallas_call(
        _head_body,
        grid=(),
        in_specs=[full(x.shape) for x in ins],
        out_specs=full((B, 1)),
        out_shape=jax.ShapeDtypeStruct((B, 1), F32),
    )(*ins)


def kernel(x, edge_index, batch, adme_features,
           Wl0, bl0, Wr0, g0, be0,
           Wl1, bl1, Wr1, g1, be1,
           Wh1, bh1, gh, beh, Wh2, bh2, Wh3, bh3):
    src2 = jnp.concatenate(
        [edge_index[0], jnp.zeros((PAD_E,), I32)]).reshape(E_UNITS, UNIT)
    dst2 = jnp.concatenate(
        [edge_index[1],
         N + (jnp.arange(PAD_E, dtype=I32) % 8)]).reshape(E_UNITS, UNIT)
    b2d = jnp.concatenate(
        [batch[:PROWS * 128],
         jnp.zeros(((PROWS_P - PROWS) * 128,), I32)]).reshape(PROWS_P, 128)
    btail = batch[PROWS * 128:].reshape(1, PTAIL)
    z2d = jnp.zeros((UNIT, L), F32)
    z1d = jnp.zeros((CQ,), F32)
    zslab = jnp.zeros((B, F), F32)
    zcnt = jnp.zeros((128,), F32)

    y0a, y0b, y0c, w0 = _proj(x, Wl0, Wr0)
    s0, cntp = _seg_sum_cnt(y0a, y0b, y0c, src2, dst2, z2d, z1d)
    pre0, stats0, cvec = _pre_stats_first(
        s0, cntp.reshape(NC, N, 1), w0, bl0.reshape(1, F))
    y1a, y1b, y1c, w1 = _bn_proj(pre0, stats0, g0.reshape(1, F),
                                 be0.reshape(1, F), Wl1, Wr1)
    s1 = _seg_sum(y1a, y1b, y1c, src2, dst2, z2d, z1d)
    if isinstance(s1, (tuple, list)):
        s1 = s1[0]
    pre1, stats1 = _pre_stats_next(s1, cvec, w1, bl1.reshape(1, F))
    h1 = _bn_relu(pre1, stats1, g1.reshape(1, F), be1.reshape(1, F))

    gsum, gcnt, gmax = _pool_sc(h1, b2d, btail, zslab, zcnt)
    out = _head(gsum, gcnt.reshape(NC, B, 1), gmax, adme_features,
                Wh1, bh1.reshape(1, 64), gh.reshape(1, 64),
                beh.reshape(1, 64), Wh2, bh2.reshape(1, 32),
                Wh3, bh3.reshape(1, 1))
    return out.reshape(B)


# speedup vs baseline: 1.0912x; 1.0912x over previous
import functools

import jax
import jax.numpy as jnp
from jax import lax
from jax.experimental import pallas as pl
from jax.experimental.pallas import tpu as pltpu
from jax.experimental.pallas import tpu_sc as plsc

F32 = jnp.float32
I32 = jnp.int32

N = 100000
E = 1600000
B = 2048
F = 48
L = 16
NCH = F // L
EPS = 1e-5

NC, NS = 2, 16
NW = NC * NS

K_SUB = 8
EROWS_P = -(-E // (128 * K_SUB * NW)) * (K_SUB * NW)
PAD_E = EROWS_P * 128 - E
ROWS_PER_W = EROWS_P // NW
N_CHUNKS = ROWS_PER_W // K_SUB
NP = N + 8

SEG_W = 6248
SEG_TAIL = N - SEG_W * NS
CNT_W = 6240
CNT_TAIL = N - CNT_W * NS
CQ = 2080

PROWS = N // 128
PTAIL = N - PROWS * 128
PROWS_P = 784
PGROUPS = PROWS_P // 8
PGPW = 3

R_TC = 2000
GRID_TC = N // R_TC

_MESH = plsc.VectorSubcoreMesh(core_axis_name="c", subcore_axis_name="s")
_SC_PARAMS = pltpu.CompilerParams(use_tc_tiling_on_sc=False)
_SC_PARAMS_POOL = pltpu.CompilerParams(use_tc_tiling_on_sc=False,
                                       needs_layout_passes=False)


def _relu(v):
    return jnp.maximum(v, 0.0)


def _dot(a, b):
    return jnp.dot(a, b, preferred_element_type=F32)


def _make_seg_sum(with_cnt):
    outs = [jax.ShapeDtypeStruct((NC, N, F), F32)]
    if with_cnt:
        outs.append(jax.ShapeDtypeStruct((NC * N,), F32))
    scratch = [
        pltpu.VMEM((2, K_SUB, 128), I32),
        pltpu.VMEM((2, K_SUB, 128), I32),
        pltpu.VMEM((K_SUB * 128, L), F32),
        pltpu.VMEM((128,), F32),
        pltpu.VMEM((CQ,), F32),
        pltpu.VMEM_SHARED((NP, L), F32),
        pltpu.SemaphoreType.DMA,
        pltpu.SemaphoreType.DMA,
    ] + [pltpu.SemaphoreType.DMA] * K_SUB
    if with_cnt:
        scratch.append(pltpu.VMEM_SHARED((NP,), F32))

    @functools.partial(pl.kernel, out_type=tuple(outs), mesh=_MESH,
                       scratch_types=scratch, compiler_params=_SC_PARAMS)
    def seg(t0, t1, t2, src_h, dst_h, z2d, z1d, *rest):
        o = rest[0]
        if with_cnt:
            ocnt = rest[1]
            sv2, dv2, rows, ones, zc, acc, ssem, isem = rest[2:10]
            gsem = rest[10:10 + K_SUB]
            cnt = rest[10 + K_SUB]
        else:
            sv2, dv2, rows, ones, zc, acc, ssem, isem = rest[1:9]
            gsem = rest[9:9 + K_SUB]
        tabs = (t0, t1, t2)
        cid = lax.axis_index("c")
        sid = lax.axis_index("s")
        wid = cid * NS + sid

        if with_cnt:
            for i in range(128 // L):
                ones[pl.ds(i * L, L)] = jnp.ones((L,), F32)

        seg_chunks = []
        off = 0
        while off < SEG_W:
            sz = min(K_SUB * 128, SEG_W - off)
            seg_chunks.append((off, sz))
            off += sz

        def edge_pipeline(tab, base, do_cnt):
            pltpu.sync_copy(src_h.at[pl.ds(base, K_SUB)], sv2.at[0])
            pltpu.sync_copy(dst_h.at[pl.ds(base, K_SUB)], dv2.at[0])

            def body(k, carry):
                p = lax.rem(k, 2)

                @pl.when(k > 0)
                def _drain_scatters():
                    pltpu.make_async_copy(z2d, rows, ssem).wait()

                @pl.when(k + 1 < N_CHUNKS)
                def _prefetch():
                    r1 = base + (k + 1) * K_SUB
                    pltpu.async_copy(src_h.at[pl.ds(r1, K_SUB)],
                                     sv2.at[1 - p], isem)
                    pltpu.async_copy(dst_h.at[pl.ds(r1, K_SUB)],
                                     dv2.at[1 - p], isem)

                descs = [
                    pltpu.async_copy(tab.at[sv2.at[p, j]],
                                     rows.at[pl.ds(j * 128, 128)], gsem[j])
                    for j in range(K_SUB)
                ]
                for j in range(K_SUB):
                    descs[j].wait()
                    pltpu.async_copy(rows.at[pl.ds(j * 128, 128)],
                                     acc.at[dv2.at[p, j]], ssem, add=True)
                    if do_cnt:
                        pltpu.sync_copy(ones, cnt.at[dv2.at[p, j]], add=True)

                @pl.when(k + 1 < N_CHUNKS)
                def _drain_prefetch():
                    pltpu.make_async_copy(src_h.at[pl.ds(0, K_SUB)],
                                          sv2.at[1 - p], isem).wait()
                    pltpu.make_async_copy(src_h.at[pl.ds(0, K_SUB)],
                                          dv2.at[1 - p], isem).wait()

                return carry

            lax.fori_loop(0, N_CHUNKS, body, 0)
            pltpu.make_async_copy(z2d, rows, ssem).wait()

        def zero_acc():
            pltpu.sync_copy(z2d, rows)
            for off, sz in seg_chunks:
                pltpu.sync_copy(rows.at[pl.ds(0, sz)],
                                acc.at[pl.ds(sid * SEG_W + off, sz)])

            @pl.when(sid == 0)
            def _zt():
                pltpu.sync_copy(rows.at[pl.ds(0, SEG_TAIL)],
                                acc.at[pl.ds(NS * SEG_W, SEG_TAIL)])

        def edge_pass(fc, do_cnt):
            edge_pipeline(tabs[fc], wid * ROWS_PER_W, do_cnt)

        zero_acc()
        if with_cnt:
            pltpu.sync_copy(z1d, zc)
            for q in range(CNT_W // CQ):
                pltpu.sync_copy(zc,
                                cnt.at[pl.ds(sid * CNT_W + q * CQ, CQ)])

            @pl.when(sid == 0)
            def _zct():
                pltpu.sync_copy(zc.at[pl.ds(0, CNT_TAIL)],
                                cnt.at[pl.ds(NS * CNT_W, CNT_TAIL)])
        plsc.subcore_barrier()

        for fc in range(NCH):
            edge_pass(fc, with_cnt and fc == 0)
            plsc.subcore_barrier()
            for off, sz in seg_chunks:
                a0 = sid * SEG_W + off
                pltpu.sync_copy(acc.at[pl.ds(a0, sz)],
                                rows.at[pl.ds(0, sz)])
                pltpu.sync_copy(rows.at[pl.ds(0, sz)],
                                o.at[cid, pl.ds(a0, sz), pl.ds(fc * L, L)])

            @pl.when(sid == 0)
            def _wtail():
                pltpu.sync_copy(acc.at[pl.ds(NS * SEG_W, SEG_TAIL)],
                                rows.at[pl.ds(0, SEG_TAIL)])
                pltpu.sync_copy(
                    rows.at[pl.ds(0, SEG_TAIL)],
                    o.at[cid, pl.ds(NS * SEG_W, SEG_TAIL), pl.ds(fc * L, L)])

            if with_cnt and fc == 0:
                for q in range(CNT_W // CQ):
                    c0 = sid * CNT_W + q * CQ
                    pltpu.sync_copy(cnt.at[pl.ds(c0, CQ)], zc)
                    pltpu.sync_copy(zc, ocnt.at[pl.ds(cid * N + c0, CQ)])

                @pl.when(sid == 0)
                def _wt():
                    pltpu.sync_copy(cnt.at[pl.ds(NS * CNT_W, CNT_TAIL)],
                                    zc.at[pl.ds(0, CNT_TAIL)])
                    pltpu.sync_copy(
                        zc.at[pl.ds(0, CNT_TAIL)],
                        ocnt.at[pl.ds(cid * N + NS * CNT_W, CNT_TAIL)])
            plsc.subcore_barrier()
            if fc + 1 < NCH:
                zero_acc()
                plsc.subcore_barrier()

    return seg


_seg_sum_cnt = _make_seg_sum(True)
_seg_sum = _make_seg_sum(False)


@functools.partial(
    pl.kernel,
    out_type=(jax.ShapeDtypeStruct((NC, B, F), F32),
              jax.ShapeDtypeStruct((NC * B,), F32),
              jax.ShapeDtypeStruct((NW, B, F), F32)),
    mesh=_MESH,
    compiler_params=_SC_PARAMS_POOL,
    scratch_types=[
        pltpu.VMEM((128, F), F32),
        pltpu.VMEM((8, 128), I32),
        pltpu.VMEM((PTAIL, F), F32),
        pltpu.VMEM((1, PTAIL), I32),
        pltpu.VMEM((128,), F32),
        pltpu.VMEM((128,), F32),
        pltpu.VMEM((B, F), F32),
        pltpu.VMEM_SHARED((B, F), F32),
        pltpu.VMEM_SHARED((B,), F32),
    ])
def _pool_sc(h_h, b2d_h, btail_h, zslab, zcnt, ogsum, ogcnt, ogmax,
             rows, ids8, rowst, idt, ones, pcbuf, slab, gsum, gcnt):
    cid = lax.axis_index("c")
    sid = lax.axis_index("s")
    wid = cid * NS + sid

    for i in range(128 // L):
        ones[pl.ds(i * L, L)] = jnp.ones((L,), F32)

    pltpu.sync_copy(zslab, slab)
    pltpu.sync_copy(zcnt, pcbuf)
    pltpu.sync_copy(slab.at[pl.ds(0, 128)], gsum.at[pl.ds(sid * 128, 128)])
    pltpu.sync_copy(pcbuf, gcnt.at[pl.ds(sid * 128, 128)])
    plsc.subcore_barrier()

    zero16 = jnp.zeros((L,), I32)

    def max_rows(rows_ref, ids_ref, idrow, nrows):
        rr8 = jnp.full((L,), idrow, I32)

        def mrow(r, carry):
            rr = jnp.full((L,), r, I32)
            bid = plsc.load_gather(ids_ref, [rr8, rr])
            for fc in range(NCH):
                col = fc * L + lax.iota(I32, L)
                v = plsc.load_gather(rows_ref, [rr, col])
                old = plsc.load_gather(slab, [bid, col])
                plsc.store_scatter(slab, [bid, col], jnp.maximum(old, v))
            return carry

        lax.fori_loop(0, nrows, mrow, 0)

    def do_group(g, nrows):
        pltpu.sync_copy(b2d_h.at[pl.ds(g * 8, 8)], ids8)
        for r8 in range(nrows):
            row = g * 8 + r8
            pltpu.sync_copy(h_h.at[pl.ds(row * 128, 128)], rows)
            pltpu.sync_copy(rows, gsum.at[ids8.at[r8]], add=True)
            pltpu.sync_copy(ones, gcnt.at[ids8.at[r8]], add=True)
            max_rows(rows, ids8, r8, 128)

    base = wid * PGPW

    def body(k, carry):
        do_group(base + k, 8)
        return carry

    lax.fori_loop(0, PGPW, body, 0)

    @pl.when(wid == 0)
    def _extra0():
        do_group(NW * PGPW, 8)

    @pl.when(wid == 1)
    def _extra1():
        do_group(NW * PGPW + 1, PROWS - (NW * PGPW + 1) * 8)

    @pl.when(wid == 0)
    def _tail():
        pltpu.sync_copy(btail_h, idt)
        pltpu.sync_copy(h_h.at[pl.ds(PROWS * 128, PTAIL)], rowst)
        pltpu.sync_copy(rowst, gsum.at[idt.at[0]], add=True)
        pltpu.sync_copy(ones.at[pl.ds(0, PTAIL)], gcnt.at[idt.at[0]],
                        add=True)
        max_rows(rowst, idt, 0, PTAIL)

    plsc.subcore_barrier()
    pltpu.sync_copy(slab, ogmax.at[wid])
    pltpu.sync_copy(gsum.at[pl.ds(sid * 128, 128)], rows)
    pltpu.sync_copy(rows, ogsum.at[cid, pl.ds(sid * 128, 128)])
    pltpu.sync_copy(gcnt.at[pl.ds(sid * 128, 128)], pcbuf)
    pltpu.sync_copy(pcbuf, ogcnt.at[pl.ds(cid * B + sid * 128, 128)])


def _proj_body(x_ref, wl_ref, wr_ref, y0_ref, y1_ref, y2_ref, w_ref):
    xb = x_ref[...]
    a = _dot(xb, wl_ref[...])
    y0_ref[...] = a[:, 0 * L:1 * L]
    y1_ref[...] = a[:, 1 * L:2 * L]
    y2_ref[...] = a[:, 2 * L:3 * L]
    w_ref[...] = _dot(xb, wr_ref[...])


def _proj(x, wl, wr):
    return pl.pallas_call(
        _proj_body,
        grid=(GRID_TC,),
        in_specs=[
            pl.BlockSpec((R_TC, F), lambda i: (i, 0)),
            pl.BlockSpec((F, F), lambda i: (0, 0)),
            pl.BlockSpec((F, F), lambda i: (0, 0)),
        ],
        out_specs=[
            pl.BlockSpec((R_TC, L), lambda i: (i, 0)),
            pl.BlockSpec((R_TC, L), lambda i: (i, 0)),
            pl.BlockSpec((R_TC, L), lambda i: (i, 0)),
            pl.BlockSpec((R_TC, F), lambda i: (i, 0)),
        ],
        out_shape=[
            jax.ShapeDtypeStruct((N, L), F32),
            jax.ShapeDtypeStruct((N, L), F32),
            jax.ShapeDtypeStruct((N, L), F32),
            jax.ShapeDtypeStruct((N, F), F32),
        ],
    )(x, wl, wr)


def _make_pre_stats(first):
    def body(*refs):
        if first:
            (s_ref, cnt_ref, w_ref, b_ref,
             pre_ref, stats_ref, cvec_ref) = refs
            c = jnp.maximum(cnt_ref[0] + cnt_ref[1], 1.0)
            cvec_ref[...] = c
        else:
            (s_ref, cvec_ref, w_ref, b_ref,
             pre_ref, stats_ref) = refs
            c = cvec_ref[...]
        s = s_ref[0] + s_ref[1]
        pre = s / c + b_ref[...] + w_ref[...]
        pre_ref[...] = pre

        @pl.when(pl.program_id(0) == 0)
        def _init():
            stats_ref[...] = jnp.zeros((2, F), F32)

        st = jnp.concatenate(
            [jnp.sum(pre, axis=0, keepdims=True),
             jnp.sum(pre * pre, axis=0, keepdims=True)], axis=0)
        stats_ref[...] += st

    sp_spec = pl.BlockSpec((NC, R_TC, F), lambda i: (0, i, 0))
    cspec = pl.BlockSpec((NC, R_TC, 1), lambda i: (0, i, 0)) if first else \
        pl.BlockSpec((R_TC, 1), lambda i: (i, 0))
    out_specs = [
        pl.BlockSpec((R_TC, F), lambda i: (i, 0)),
        pl.BlockSpec((2, F), lambda i: (0, 0)),
    ]
    out_shape = [
        jax.ShapeDtypeStruct((N, F), F32),
        jax.ShapeDtypeStruct((2, F), F32),
    ]
    if first:
        out_specs.append(pl.BlockSpec((R_TC, 1), lambda i: (i, 0)))
        out_shape.append(jax.ShapeDtypeStruct((N, 1), F32))

    def run(s, cdata, w, bias):
        return pl.pallas_call(
            body,
            grid=(GRID_TC,),
            in_specs=[sp_spec, cspec,
                      pl.BlockSpec((R_TC, F), lambda i: (i, 0)),
                      pl.BlockSpec((1, F), lambda i: (0, 0))],
            out_specs=out_specs,
            out_shape=out_shape,
        )(s, cdata, w, bias)

    return run


_pre_stats_first = _make_pre_stats(True)
_pre_stats_next = _make_pre_stats(False)


def _bn_common(pre_ref, stats_ref, g_ref, be_ref):
    st = stats_ref[...]
    mean = st[0:1, :] * (1.0 / N)
    var = st[1:2, :] * (1.0 / N) - mean * mean
    rstd = lax.rsqrt(var + EPS)
    return _relu((pre_ref[...] - mean) * rstd * g_ref[...] + be_ref[...])


def _bn_proj_body(pre_ref, stats_ref, g_ref, be_ref, wl_ref, wr_ref,
                  y0_ref, y1_ref, y2_ref, w_ref):
    h = _bn_common(pre_ref, stats_ref, g_ref, be_ref)
    a = _dot(h, wl_ref[...])
    y0_ref[...] = a[:, 0 * L:1 * L]
    y1_ref[...] = a[:, 1 * L:2 * L]
    y2_ref[...] = a[:, 2 * L:3 * L]
    w_ref[...] = _dot(h, wr_ref[...])


def _bn_proj(pre, stats, g, be, wl, wr):
    return pl.pallas_call(
        _bn_proj_body,
        grid=(GRID_TC,),
        in_specs=[
            pl.BlockSpec((R_TC, F), lambda i: (i, 0)),
            pl.BlockSpec((2, F), lambda i: (0, 0)),
            pl.BlockSpec((1, F), lambda i: (0, 0)),
            pl.BlockSpec((1, F), lambda i: (0, 0)),
            pl.BlockSpec((F, F), lambda i: (0, 0)),
            pl.BlockSpec((F, F), lambda i: (0, 0)),
        ],
        out_specs=[
            pl.BlockSpec((R_TC, L), lambda i: (i, 0)),
            pl.BlockSpec((R_TC, L), lambda i: (i, 0)),
            pl.BlockSpec((R_TC, L), lambda i: (i, 0)),
            pl.BlockSpec((R_TC, F), lambda i: (i, 0)),
        ],
        out_shape=[
            jax.ShapeDtypeStruct((N, L), F32),
            jax.ShapeDtypeStruct((N, L), F32),
            jax.ShapeDtypeStruct((N, L), F32),
            jax.ShapeDtypeStruct((N, F), F32),
        ],
    )(pre, stats, g, be, wl, wr)


def _bn_relu_body(pre_ref, stats_ref, g_ref, be_ref, h_ref):
    h_ref[...] = _bn_common(pre_ref, stats_ref, g_ref, be_ref)


def _bn_relu(pre, stats, g, be):
    return pl.pallas_call(
        _bn_relu_body,
        grid=(GRID_TC,),
        in_specs=[
            pl.BlockSpec((R_TC, F), lambda i: (i, 0)),
            pl.BlockSpec((2, F), lambda i: (0, 0)),
            pl.BlockSpec((1, F), lambda i: (0, 0)),
            pl.BlockSpec((1, F), lambda i: (0, 0)),
        ],
        out_specs=pl.BlockSpec((R_TC, F), lambda i: (i, 0)),
        out_shape=jax.ShapeDtypeStruct((N, F), F32),
    )(pre, stats, g, be)


def _head_body(gsum_ref, gcnt_ref, gmax_ref, adme_ref,
               wh1_ref, bh1_ref, gh_ref, beh_ref, wh2_ref, bh2_ref,
               wh3_ref, bh3_ref, out_ref):
    gsum = gsum_ref[0] + gsum_ref[1]
    gcnt = jnp.maximum(gcnt_ref[0] + gcnt_ref[1], 1.0)
    gmean = gsum / gcnt
    m = gmax_ref[0]
    for i in range(1, NW):
        m = jnp.maximum(m, gmax_ref[i])
    comb = jnp.concatenate([gmean, m, adme_ref[...]], axis=-1)
    z = _dot(comb, wh1_ref[...]) + bh1_ref[...]
    mean = jnp.mean(z, axis=0, keepdims=True)
    var = jnp.mean(z * z, axis=0, keepdims=True) - mean * mean
    z = _relu((z - mean) * lax.rsqrt(var + EPS) * gh_ref[...] + beh_ref[...])
    z = _relu(_dot(z, wh2_ref[...]) + bh2_ref[...])
    out_ref[...] = _dot(z, wh3_ref[...]) + bh3_ref[...]


def _head(gsum, gcnt, gmax, adme, wh1, bh1, gh, beh, wh2, bh2, wh3, bh3):
    full = lambda shape: pl.BlockSpec(shape, lambda: tuple(0 for _ in shape))
    ins = [gsum, gcnt, gmax, adme, wh1, bh1, gh, beh, wh2, bh2, wh3, bh3]
    return pl.pallas_call(
        _head_body,
        grid=(),
        in_specs=[full(x.shape) for x in ins],
        out_specs=full((B, 1)),
        out_shape=jax.ShapeDtypeStruct((B, 1), F32),
    )(*ins)


def kernel(x, edge_index, batch, adme_features,
           Wl0, bl0, Wr0, g0, be0,
           Wl1, bl1, Wr1, g1, be1,
           Wh1, bh1, gh, beh, Wh2, bh2, Wh3, bh3):
    src2 = jnp.concatenate(
        [edge_index[0], jnp.zeros((PAD_E,), I32)]).reshape(EROWS_P, 128)
    dst2 = jnp.concatenate(
        [edge_index[1],
         N + (jnp.arange(PAD_E, dtype=I32) % 8)]).reshape(EROWS_P, 128)
    b2d = jnp.concatenate(
        [batch[:PROWS * 128],
         jnp.zeros(((PROWS_P - PROWS) * 128,), I32)]).reshape(PROWS_P, 128)
    btail = batch[PROWS * 128:].reshape(1, PTAIL)
    z2d = jnp.zeros((K_SUB * 128, L), F32)
    z1d = jnp.zeros((CQ,), F32)
    zslab = jnp.zeros((B, F), F32)
    zcnt = jnp.zeros((128,), F32)

    y0a, y0b, y0c, w0 = _proj(x, Wl0, Wr0)
    s0, cntp = _seg_sum_cnt(y0a, y0b, y0c, src2, dst2, z2d, z1d)
    pre0, stats0, cvec = _pre_stats_first(
        s0, cntp.reshape(NC, N, 1), w0, bl0.reshape(1, F))
    y1a, y1b, y1c, w1 = _bn_proj(pre0, stats0, g0.reshape(1, F),
                                 be0.reshape(1, F), Wl1, Wr1)
    s1 = _seg_sum(y1a, y1b, y1c, src2, dst2, z2d, z1d)
    if isinstance(s1, (tuple, list)):
        s1 = s1[0]
    pre1, stats1 = _pre_stats_next(s1, cvec, w1, bl1.reshape(1, F))
    h1 = _bn_relu(pre1, stats1, g1.reshape(1, F), be1.reshape(1, F))

    gsum, gcnt, gmax = _pool_sc(h1, b2d, btail, zslab, zcnt)
    out = _head(gsum, gcnt.reshape(NC, B, 1), gmax, adme_features,
                Wh1, bh1.reshape(1, 64), gh.reshape(1, 64),
                beh.reshape(1, 64), Wh2, bh2.reshape(1, 32),
                Wh3, bh3.reshape(1, 1))
    return out.reshape(B)


# speedup vs baseline: 1.1104x; 1.0176x over previous
import functools

import jax
import jax.numpy as jnp
from jax import lax
from jax.experimental import pallas as pl
from jax.experimental.pallas import tpu as pltpu
from jax.experimental.pallas import tpu_sc as plsc

F32 = jnp.float32
I32 = jnp.int32

N = 100000
E = 1600000
B = 2048
F = 48
L = 16
NCH = F // L
EPS = 1e-5

NC, NS = 2, 16
NW = NC * NS

K_SUB = 8
EROWS_P = -(-E // (128 * K_SUB * NW)) * (K_SUB * NW)
PAD_E = EROWS_P * 128 - E
ROWS_PER_W = EROWS_P // NW
N_CHUNKS = ROWS_PER_W // K_SUB
NP = N + 8

SEG_W = 6248
SEG_TAIL = N - SEG_W * NS
CNT_W = 6240
CNT_TAIL = N - CNT_W * NS
CQ = 2080

PROWS = N // 128
PTAIL = N - PROWS * 128
PROWS_P = 784
PGROUPS = PROWS_P // 8
PGPW = 3

R_TC = 4000
GRID_TC = N // R_TC

_MESH = plsc.VectorSubcoreMesh(core_axis_name="c", subcore_axis_name="s")
_SC_PARAMS = pltpu.CompilerParams(use_tc_tiling_on_sc=False)
_SC_PARAMS_POOL = pltpu.CompilerParams(use_tc_tiling_on_sc=False,
                                       needs_layout_passes=False)


def _relu(v):
    return jnp.maximum(v, 0.0)


def _dot(a, b):
    return jnp.dot(a, b, preferred_element_type=F32)


def _make_seg_sum(with_cnt):
    outs = [jax.ShapeDtypeStruct((NC, N, F), F32)]
    if with_cnt:
        outs.append(jax.ShapeDtypeStruct((NC * N,), F32))
    scratch = [
        pltpu.VMEM((2, K_SUB, 128), I32),
        pltpu.VMEM((2, K_SUB, 128), I32),
        pltpu.VMEM((K_SUB * 128, L), F32),
        pltpu.VMEM((128,), F32),
        pltpu.VMEM((CQ,), F32),
        pltpu.VMEM_SHARED((NP, L), F32),
        pltpu.SemaphoreType.DMA,
        pltpu.SemaphoreType.DMA,
    ] + [pltpu.SemaphoreType.DMA] * K_SUB
    if with_cnt:
        scratch.append(pltpu.VMEM_SHARED((NP,), F32))

    @functools.partial(pl.kernel, out_type=tuple(outs), mesh=_MESH,
                       scratch_types=scratch, compiler_params=_SC_PARAMS)
    def seg(t0, t1, t2, src_h, dst_h, z2d, z1d, *rest):
        o = rest[0]
        if with_cnt:
            ocnt = rest[1]
            sv2, dv2, rows, ones, zc, acc, ssem, isem = rest[2:10]
            gsem = rest[10:10 + K_SUB]
            cnt = rest[10 + K_SUB]
        else:
            sv2, dv2, rows, ones, zc, acc, ssem, isem = rest[1:9]
            gsem = rest[9:9 + K_SUB]
        tabs = (t0, t1, t2)
        cid = lax.axis_index("c")
        sid = lax.axis_index("s")
        wid = cid * NS + sid

        if with_cnt:
            for i in range(128 // L):
                ones[pl.ds(i * L, L)] = jnp.ones((L,), F32)

        seg_chunks = []
        off = 0
        while off < SEG_W:
            sz = min(K_SUB * 128, SEG_W - off)
            seg_chunks.append((off, sz))
            off += sz

        def edge_pipeline(tab, base, do_cnt):
            pltpu.sync_copy(src_h.at[pl.ds(base, K_SUB)], sv2.at[0])
            pltpu.sync_copy(dst_h.at[pl.ds(base, K_SUB)], dv2.at[0])

            def body(k, carry):
                p = lax.rem(k, 2)

                @pl.when(k > 0)
                def _drain_scatters():
                    pltpu.make_async_copy(z2d, rows, ssem).wait()

                @pl.when(k + 1 < N_CHUNKS)
                def _prefetch():
                    r1 = base + (k + 1) * K_SUB
                    pltpu.async_copy(src_h.at[pl.ds(r1, K_SUB)],
                                     sv2.at[1 - p], isem)
                    pltpu.async_copy(dst_h.at[pl.ds(r1, K_SUB)],
                                     dv2.at[1 - p], isem)

                descs = [
                    pltpu.async_copy(tab.at[sv2.at[p, j]],
                                     rows.at[pl.ds(j * 128, 128)], gsem[j])
                    for j in range(K_SUB)
                ]
                for j in range(K_SUB):
                    descs[j].wait()
                    pltpu.async_copy(rows.at[pl.ds(j * 128, 128)],
                                     acc.at[dv2.at[p, j]], ssem, add=True)
                    if do_cnt:
                        pltpu.sync_copy(ones, cnt.at[dv2.at[p, j]], add=True)

                @pl.when(k + 1 < N_CHUNKS)
                def _drain_prefetch():
                    pltpu.make_async_copy(src_h.at[pl.ds(0, K_SUB)],
                                          sv2.at[1 - p], isem).wait()
                    pltpu.make_async_copy(src_h.at[pl.ds(0, K_SUB)],
                                          dv2.at[1 - p], isem).wait()

                return carry

            lax.fori_loop(0, N_CHUNKS, body, 0)
            pltpu.make_async_copy(z2d, rows, ssem).wait()

        def zero_acc():
            pltpu.sync_copy(z2d, rows)
            for off, sz in seg_chunks:
                pltpu.sync_copy(rows.at[pl.ds(0, sz)],
                                acc.at[pl.ds(sid * SEG_W + off, sz)])

            @pl.when(sid == 0)
            def _zt():
                pltpu.sync_copy(rows.at[pl.ds(0, SEG_TAIL)],
                                acc.at[pl.ds(NS * SEG_W, SEG_TAIL)])

        def edge_pass(fc, do_cnt):
            edge_pipeline(tabs[fc], wid * ROWS_PER_W, do_cnt)

        zero_acc()
        if with_cnt:
            pltpu.sync_copy(z1d, zc)
            for q in range(CNT_W // CQ):
                pltpu.sync_copy(zc,
                                cnt.at[pl.ds(sid * CNT_W + q * CQ, CQ)])

            @pl.when(sid == 0)
            def _zct():
                pltpu.sync_copy(zc.at[pl.ds(0, CNT_TAIL)],
                                cnt.at[pl.ds(NS * CNT_W, CNT_TAIL)])
        plsc.subcore_barrier()

        for fc in range(NCH):
            edge_pass(fc, with_cnt and fc == 0)
            plsc.subcore_barrier()
            for off, sz in seg_chunks:
                a0 = sid * SEG_W + off
                pltpu.sync_copy(acc.at[pl.ds(a0, sz)],
                                rows.at[pl.ds(0, sz)])
                pltpu.sync_copy(rows.at[pl.ds(0, sz)],
                                o.at[cid, pl.ds(a0, sz), pl.ds(fc * L, L)])

            @pl.when(sid == 0)
            def _wtail():
                pltpu.sync_copy(acc.at[pl.ds(NS * SEG_W, SEG_TAIL)],
                                rows.at[pl.ds(0, SEG_TAIL)])
                pltpu.sync_copy(
                    rows.at[pl.ds(0, SEG_TAIL)],
                    o.at[cid, pl.ds(NS * SEG_W, SEG_TAIL), pl.ds(fc * L, L)])

            if with_cnt and fc == 0:
                for q in range(CNT_W // CQ):
                    c0 = sid * CNT_W + q * CQ
                    pltpu.sync_copy(cnt.at[pl.ds(c0, CQ)], zc)
                    pltpu.sync_copy(zc, ocnt.at[pl.ds(cid * N + c0, CQ)])

                @pl.when(sid == 0)
                def _wt():
                    pltpu.sync_copy(cnt.at[pl.ds(NS * CNT_W, CNT_TAIL)],
                                    zc.at[pl.ds(0, CNT_TAIL)])
                    pltpu.sync_copy(
                        zc.at[pl.ds(0, CNT_TAIL)],
                        ocnt.at[pl.ds(cid * N + NS * CNT_W, CNT_TAIL)])
            plsc.subcore_barrier()
            if fc + 1 < NCH:
                zero_acc()
                plsc.subcore_barrier()

    return seg


_seg_sum_cnt = _make_seg_sum(True)
_seg_sum = _make_seg_sum(False)


@functools.partial(
    pl.kernel,
    out_type=(jax.ShapeDtypeStruct((NC, B, F), F32),
              jax.ShapeDtypeStruct((NC * B,), F32),
              jax.ShapeDtypeStruct((NW, B, F), F32)),
    mesh=_MESH,
    compiler_params=_SC_PARAMS_POOL,
    scratch_types=[
        pltpu.VMEM((128, F), F32),
        pltpu.VMEM((8, 128), I32),
        pltpu.VMEM((PTAIL, F), F32),
        pltpu.VMEM((1, PTAIL), I32),
        pltpu.VMEM((128,), F32),
        pltpu.VMEM((128,), F32),
        pltpu.VMEM((B, F), F32),
        pltpu.VMEM_SHARED((B, F), F32),
        pltpu.VMEM_SHARED((B,), F32),
    ])
def _pool_sc(h_h, b2d_h, btail_h, zslab, zcnt, ogsum, ogcnt, ogmax,
             rows, ids8, rowst, idt, ones, pcbuf, slab, gsum, gcnt):
    cid = lax.axis_index("c")
    sid = lax.axis_index("s")
    wid = cid * NS + sid

    for i in range(128 // L):
        ones[pl.ds(i * L, L)] = jnp.ones((L,), F32)

    pltpu.sync_copy(zslab, slab)
    pltpu.sync_copy(zcnt, pcbuf)
    pltpu.sync_copy(slab.at[pl.ds(0, 128)], gsum.at[pl.ds(sid * 128, 128)])
    pltpu.sync_copy(pcbuf, gcnt.at[pl.ds(sid * 128, 128)])
    plsc.subcore_barrier()

    zero16 = jnp.zeros((L,), I32)

    def max_rows(rows_ref, ids_ref, idrow, nrows):
        rr8 = jnp.full((L,), idrow, I32)

        def mrow(r, carry):
            rr = jnp.full((L,), r, I32)
            bid = plsc.load_gather(ids_ref, [rr8, rr])
            for fc in range(NCH):
                col = fc * L + lax.iota(I32, L)
                v = plsc.load_gather(rows_ref, [rr, col])
                old = plsc.load_gather(slab, [bid, col])
                plsc.store_scatter(slab, [bid, col], jnp.maximum(old, v))
            return carry

        lax.fori_loop(0, nrows, mrow, 0)

    def do_group(g, nrows):
        pltpu.sync_copy(b2d_h.at[pl.ds(g * 8, 8)], ids8)
        for r8 in range(nrows):
            row = g * 8 + r8
            pltpu.sync_copy(h_h.at[pl.ds(row * 128, 128)], rows)
            pltpu.sync_copy(rows, gsum.at[ids8.at[r8]], add=True)
            pltpu.sync_copy(ones, gcnt.at[ids8.at[r8]], add=True)
            max_rows(rows, ids8, r8, 128)

    base = wid * PGPW

    def body(k, carry):
        do_group(base + k, 8)
        return carry

    lax.fori_loop(0, PGPW, body, 0)

    @pl.when(wid == 0)
    def _extra0():
        do_group(NW * PGPW, 8)

    @pl.when(wid == 1)
    def _extra1():
        do_group(NW * PGPW + 1, PROWS - (NW * PGPW + 1) * 8)

    @pl.when(wid == 0)
    def _tail():
        pltpu.sync_copy(btail_h, idt)
        pltpu.sync_copy(h_h.at[pl.ds(PROWS * 128, PTAIL)], rowst)
        pltpu.sync_copy(rowst, gsum.at[idt.at[0]], add=True)
        pltpu.sync_copy(ones.at[pl.ds(0, PTAIL)], gcnt.at[idt.at[0]],
                        add=True)
        max_rows(rowst, idt, 0, PTAIL)

    plsc.subcore_barrier()
    pltpu.sync_copy(slab, ogmax.at[wid])
    pltpu.sync_copy(gsum.at[pl.ds(sid * 128, 128)], rows)
    pltpu.sync_copy(rows, ogsum.at[cid, pl.ds(sid * 128, 128)])
    pltpu.sync_copy(gcnt.at[pl.ds(sid * 128, 128)], pcbuf)
    pltpu.sync_copy(pcbuf, ogcnt.at[pl.ds(cid * B + sid * 128, 128)])


def _proj_body(x_ref, wl_ref, wr_ref, y0_ref, y1_ref, y2_ref, w_ref):
    xb = x_ref[...]
    a = _dot(xb, wl_ref[...])
    y0_ref[...] = a[:, 0 * L:1 * L]
    y1_ref[...] = a[:, 1 * L:2 * L]
    y2_ref[...] = a[:, 2 * L:3 * L]
    w_ref[...] = _dot(xb, wr_ref[...])


def _proj(x, wl, wr):
    return pl.pallas_call(
        _proj_body,
        grid=(GRID_TC,),
        in_specs=[
            pl.BlockSpec((R_TC, F), lambda i: (i, 0)),
            pl.BlockSpec((F, F), lambda i: (0, 0)),
            pl.BlockSpec((F, F), lambda i: (0, 0)),
        ],
        out_specs=[
            pl.BlockSpec((R_TC, L), lambda i: (i, 0)),
            pl.BlockSpec((R_TC, L), lambda i: (i, 0)),
            pl.BlockSpec((R_TC, L), lambda i: (i, 0)),
            pl.BlockSpec((R_TC, F), lambda i: (i, 0)),
        ],
        out_shape=[
            jax.ShapeDtypeStruct((N, L), F32),
            jax.ShapeDtypeStruct((N, L), F32),
            jax.ShapeDtypeStruct((N, L), F32),
            jax.ShapeDtypeStruct((N, F), F32),
        ],
    )(x, wl, wr)


def _make_pre_stats(first):
    def body(*refs):
        if first:
            (s_ref, cnt_ref, w_ref, b_ref,
             pre_ref, stats_ref, cvec_ref) = refs
            c = jnp.maximum(cnt_ref[0] + cnt_ref[1], 1.0)
            cvec_ref[...] = c
        else:
            (s_ref, cvec_ref, w_ref, b_ref,
             pre_ref, stats_ref) = refs
            c = cvec_ref[...]
        s = s_ref[0] + s_ref[1]
        pre = s / c + b_ref[...] + w_ref[...]
        pre_ref[...] = pre

        @pl.when(pl.program_id(0) == 0)
        def _init():
            stats_ref[...] = jnp.zeros((2, F), F32)

        st = jnp.concatenate(
            [jnp.sum(pre, axis=0, keepdims=True),
             jnp.sum(pre * pre, axis=0, keepdims=True)], axis=0)
        stats_ref[...] += st

    sp_spec = pl.BlockSpec((NC, R_TC, F), lambda i: (0, i, 0))
    cspec = pl.BlockSpec((NC, R_TC, 1), lambda i: (0, i, 0)) if first else \
        pl.BlockSpec((R_TC, 1), lambda i: (i, 0))
    out_specs = [
        pl.BlockSpec((R_TC, F), lambda i: (i, 0)),
        pl.BlockSpec((2, F), lambda i: (0, 0)),
    ]
    out_shape = [
        jax.ShapeDtypeStruct((N, F), F32),
        jax.ShapeDtypeStruct((2, F), F32),
    ]
    if first:
        out_specs.append(pl.BlockSpec((R_TC, 1), lambda i: (i, 0)))
        out_shape.append(jax.ShapeDtypeStruct((N, 1), F32))

    def run(s, cdata, w, bias):
        return pl.pallas_call(
            body,
            grid=(GRID_TC,),
            in_specs=[sp_spec, cspec,
                      pl.BlockSpec((R_TC, F), lambda i: (i, 0)),
                      pl.BlockSpec((1, F), lambda i: (0, 0))],
            out_specs=out_specs,
            out_shape=out_shape,
        )(s, cdata, w, bias)

    return run


_pre_stats_first = _make_pre_stats(True)
_pre_stats_next = _make_pre_stats(False)


def _bn_common(pre_ref, stats_ref, g_ref, be_ref):
    st = stats_ref[...]
    mean = st[0:1, :] * (1.0 / N)
    var = st[1:2, :] * (1.0 / N) - mean * mean
    rstd = lax.rsqrt(var + EPS)
    return _relu((pre_ref[...] - mean) * rstd * g_ref[...] + be_ref[...])


def _bn_proj_body(pre_ref, stats_ref, g_ref, be_ref, wl_ref, wr_ref,
                  y0_ref, y1_ref, y2_ref, w_ref):
    h = _bn_common(pre_ref, stats_ref, g_ref, be_ref)
    a = _dot(h, wl_ref[...])
    y0_ref[...] = a[:, 0 * L:1 * L]
    y1_ref[...] = a[:, 1 * L:2 * L]
    y2_ref[...] = a[:, 2 * L:3 * L]
    w_ref[...] = _dot(h, wr_ref[...])


def _bn_proj(pre, stats, g, be, wl, wr):
    return pl.pallas_call(
        _bn_proj_body,
        grid=(GRID_TC,),
        in_specs=[
            pl.BlockSpec((R_TC, F), lambda i: (i, 0)),
            pl.BlockSpec((2, F), lambda i: (0, 0)),
            pl.BlockSpec((1, F), lambda i: (0, 0)),
            pl.BlockSpec((1, F), lambda i: (0, 0)),
            pl.BlockSpec((F, F), lambda i: (0, 0)),
            pl.BlockSpec((F, F), lambda i: (0, 0)),
        ],
        out_specs=[
            pl.BlockSpec((R_TC, L), lambda i: (i, 0)),
            pl.BlockSpec((R_TC, L), lambda i: (i, 0)),
            pl.BlockSpec((R_TC, L), lambda i: (i, 0)),
            pl.BlockSpec((R_TC, F), lambda i: (i, 0)),
        ],
        out_shape=[
            jax.ShapeDtypeStruct((N, L), F32),
            jax.ShapeDtypeStruct((N, L), F32),
            jax.ShapeDtypeStruct((N, L), F32),
            jax.ShapeDtypeStruct((N, F), F32),
        ],
    )(pre, stats, g, be, wl, wr)


def _bn_relu_body(pre_ref, stats_ref, g_ref, be_ref, h_ref):
    h_ref[...] = _bn_common(pre_ref, stats_ref, g_ref, be_ref)


def _bn_relu(pre, stats, g, be):
    return pl.pallas_call(
        _bn_relu_body,
        grid=(GRID_TC,),
        in_specs=[
            pl.BlockSpec((R_TC, F), lambda i: (i, 0)),
            pl.BlockSpec((2, F), lambda i: (0, 0)),
            pl.BlockSpec((1, F), lambda i: (0, 0)),
            pl.BlockSpec((1, F), lambda i: (0, 0)),
        ],
        out_specs=pl.BlockSpec((R_TC, F), lambda i: (i, 0)),
        out_shape=jax.ShapeDtypeStruct((N, F), F32),
    )(pre, stats, g, be)


def _head_body(gsum_ref, gcnt_ref, gmax_ref, adme_ref,
               wh1_ref, bh1_ref, gh_ref, beh_ref, wh2_ref, bh2_ref,
               wh3_ref, bh3_ref, out_ref):
    gsum = gsum_ref[0] + gsum_ref[1]
    gcnt = jnp.maximum(gcnt_ref[0] + gcnt_ref[1], 1.0)
    gmean = gsum / gcnt
    m = gmax_ref[0]
    for i in range(1, NW):
        m = jnp.maximum(m, gmax_ref[i])
    comb = jnp.concatenate([gmean, m, adme_ref[...]], axis=-1)
    z = _dot(comb, wh1_ref[...]) + bh1_ref[...]
    mean = jnp.mean(z, axis=0, keepdims=True)
    var = jnp.mean(z * z, axis=0, keepdims=True) - mean * mean
    z = _relu((z - mean) * lax.rsqrt(var + EPS) * gh_ref[...] + beh_ref[...])
    z = _relu(_dot(z, wh2_ref[...]) + bh2_ref[...])
    out_ref[...] = _dot(z, wh3_ref[...]) + bh3_ref[...]


def _head(gsum, gcnt, gmax, adme, wh1, bh1, gh, beh, wh2, bh2, wh3, bh3):
    full = lambda shape: pl.BlockSpec(shape, lambda: tuple(0 for _ in shape))
    ins = [gsum, gcnt, gmax, adme, wh1, bh1, gh, beh, wh2, bh2, wh3, bh3]
    return pl.pallas_call(
        _head_body,
        grid=(),
        in_specs=[full(x.shape) for x in ins],
        out_specs=full((B, 1)),
        out_shape=jax.ShapeDtypeStruct((B, 1), F32),
    )(*ins)


def kernel(x, edge_index, batch, adme_features,
           Wl0, bl0, Wr0, g0, be0,
           Wl1, bl1, Wr1, g1, be1,
           Wh1, bh1, gh, beh, Wh2, bh2, Wh3, bh3):
    src2 = jnp.concatenate(
        [edge_index[0], jnp.zeros((PAD_E,), I32)]).reshape(EROWS_P, 128)
    dst2 = jnp.concatenate(
        [edge_index[1],
         N + (jnp.arange(PAD_E, dtype=I32) % 8)]).reshape(EROWS_P, 128)
    b2d = jnp.concatenate(
        [batch[:PROWS * 128],
         jnp.zeros(((PROWS_P - PROWS) * 128,), I32)]).reshape(PROWS_P, 128)
    btail = batch[PROWS * 128:].reshape(1, PTAIL)
    z2d = jnp.zeros((K_SUB * 128, L), F32)
    z1d = jnp.zeros((CQ,), F32)
    zslab = jnp.zeros((B, F), F32)
    zcnt = jnp.zeros((128,), F32)

    y0a, y0b, y0c, w0 = _proj(x, Wl0, Wr0)
    s0, cntp = _seg_sum_cnt(y0a, y0b, y0c, src2, dst2, z2d, z1d)
    pre0, stats0, cvec = _pre_stats_first(
        s0, cntp.reshape(NC, N, 1), w0, bl0.reshape(1, F))
    y1a, y1b, y1c, w1 = _bn_proj(pre0, stats0, g0.reshape(1, F),
                                 be0.reshape(1, F), Wl1, Wr1)
    s1 = _seg_sum(y1a, y1b, y1c, src2, dst2, z2d, z1d)
    if isinstance(s1, (tuple, list)):
        s1 = s1[0]
    pre1, stats1 = _pre_stats_next(s1, cvec, w1, bl1.reshape(1, F))
    h1 = _bn_relu(pre1, stats1, g1.reshape(1, F), be1.reshape(1, F))

    gsum, gcnt, gmax = _pool_sc(h1, b2d, btail, zslab, zcnt)
    out = _head(gsum, gcnt.reshape(NC, B, 1), gmax, adme_features,
                Wh1, bh1.reshape(1, 64), gh.reshape(1, 64),
                beh.reshape(1, 64), Wh2, bh2.reshape(1, 32),
                Wh3, bh3.reshape(1, 1))
    return out.reshape(B)
